# baseline (speedup 1.0000x reference)
import functools

import jax
import jax.numpy as jnp
from jax import lax
from jax.experimental import pallas as pl
from jax.experimental.pallas import tpu as pltpu
from jax.experimental.pallas import tpu_sc as plsc

GRID_N = 256
NPTS = 8192 * 128
NW = 32
PER_W = NPTS // NW
BC = 1024
NB = PER_W // BC
ROW = 128
NR = BC // ROW
L = 16
TABLE_W = 32 * 16384
MAGIC = 12582912.0


def _pack_body(c_ref, out_ref):
    x = c_ref[...]
    b = lax.broadcasted_iota(jnp.int32, (8, 1, 1), 0)
    out_ref[...] = jnp.sum(x << b, axis=0)


def _pack(comb32):
    return pl.pallas_call(
        _pack_body,
        grid=(16,),
        in_specs=[
            pl.BlockSpec((8, 32, 1024), lambda g: (0, 0, g)),
        ],
        out_specs=pl.BlockSpec((32, 1024), lambda g: (0, g)),
        out_shape=jax.ShapeDtypeStruct((32, 16384), jnp.int32),
    )(comb32)


def _sc_body(xyz_hbm, table_hbm, params_hbm, out_hbm,
             pv, xall, wv, av, gv, ov, tsh, insem, gsem):
    sid = lax.axis_index("s")
    wid = sid * 2 + lax.axis_index("c")
    base_pt = wid * PER_W

    TW = TABLE_W // 16
    pltpu.sync_copy(table_hbm.at[pl.ds(sid * TW, TW)],
                    tsh.at[pl.ds(sid * TW, TW)])
    plsc.subcore_barrier()

    pltpu.sync_copy(params_hbm, pv)
    sx = pv[pl.ds(0, L)]
    sy = pv[pl.ds(L, L)]
    sz = pv[pl.ds(2 * L, L)]
    tx = pv[pl.ds(3 * L, L)]
    ty = pv[pl.ds(4 * L, L)]
    tz = pv[pl.ds(5 * L, L)]

    for c in range(3):
        pltpu.async_copy(xyz_hbm.at[pl.ds(c * NPTS + base_pt, BC)],
                         xall.at[pl.ds(c * BC, BC)], insem)

    def block_body(t, carry):
        pt0 = base_pt + t * BC
        pltpu.make_async_copy(xyz_hbm.at[pl.ds(0, 3 * BC)], xall, insem).wait()

        for r in range(NR):
            for gg in range(ROW // L):
                off = r * ROW + gg * L
                xs = xall[pl.ds(off, L)]
                ys = xall[pl.ds(BC + off, L)]
                zs = xall[pl.ds(2 * BC + off, L)]

                fx = (xs * sx + tx + MAGIC) - MAGIC
                fy = (ys * sy + ty + MAGIC) - MAGIC
                fz = (zs * sz + tz + MAGIC) - MAGIC

                ii = fx.astype(jnp.int32)
                jj = fy.astype(jnp.int32)
                kk = fz.astype(jnp.int32)

                inb = ((ii | jj | kk) & -256) == 0
                my = jj & 255
                mz = kk & 255

                wv[pl.ds(off, L)] = (
                    ((ii & 31) << 14) | (my << 6)
                    | lax.shift_right_logical(mz, 2))
                av[pl.ds(off, L)] = (
                    ((mz & 3) << 3)
                    | (lax.shift_right_logical(ii, 5) & 7)
                    | jnp.where(inb, 32, 0))

            pltpu.async_copy(tsh.at[wv.at[pl.ds(r * ROW, ROW)]],
                             gv.at[pl.ds(r * ROW, ROW)], gsem)

        ptn = base_pt + lax.rem(t + 1, NB) * BC
        for c in range(3):
            pltpu.async_copy(xyz_hbm.at[pl.ds(c * NPTS + ptn, BC)],
                             xall.at[pl.ds(c * BC, BC)], insem)

        pltpu.make_async_copy(table_hbm.at[pl.ds(0, BC)], gv, gsem).wait()

        for g in range(BC // L):
            w = gv[pl.ds(g * L, L)]
            a = av[pl.ds(g * L, L)]
            val = (lax.shift_right_logical(w, a & 31) & 1) \
                & lax.shift_right_logical(a, 5)
            ov[pl.ds(g * L, L)] = val

        pltpu.sync_copy(ov, out_hbm.at[pl.ds(pt0, BC)])
        return carry

    lax.fori_loop(0, NB, block_body, 0)
    pltpu.make_async_copy(xyz_hbm.at[pl.ds(0, 3 * BC)], xall, insem).wait()


@functools.partial(
    pl.kernel,
    out_type=jax.ShapeDtypeStruct((NPTS,), jnp.int32),
    mesh=plsc.VectorSubcoreMesh(core_axis_name="c", subcore_axis_name="s"),
    compiler_params=pltpu.CompilerParams(needs_layout_passes=False),
    scratch_types=[
        pltpu.VMEM((6 * L,), jnp.float32),
        pltpu.VMEM((3 * BC,), jnp.float32),
        pltpu.VMEM((BC,), jnp.int32),
        pltpu.VMEM((BC,), jnp.int32),
        pltpu.VMEM((BC,), jnp.int32),
        pltpu.VMEM((BC,), jnp.int32),
        pltpu.VMEM_SHARED((TABLE_W,), jnp.int32),
        pltpu.SemaphoreType.DMA,
        pltpu.SemaphoreType.DMA,
    ],
)
def _lookup(xyz_hbm, table_hbm, params_hbm, out_hbm, *scratch):
    _sc_body(xyz_hbm, table_hbm, params_hbm, out_hbm, *scratch)


def kernel(xyz, mask, bound_mask, xyz2ijk_scale, xyz2ijk_shift):
    shape = xyz.shape[:-1]
    comb_u8 = jnp.logical_and(mask, bound_mask).astype(jnp.uint8)
    comb32 = comb_u8.view(jnp.int32).reshape(8, 32, 16384)
    packed = _pack(comb32).reshape(-1)
    xflat = jnp.transpose(xyz, (2, 0, 1)).reshape(-1)
    params = jnp.concatenate([
        jnp.repeat(xyz2ijk_scale.astype(jnp.float32), L),
        jnp.repeat(xyz2ijk_shift.astype(jnp.float32), L),
    ])
    flat = _lookup(xflat, packed, params)
    return flat.astype(jnp.bool_).reshape(shape)


# speedup vs baseline: 1.6978x; 1.6978x over previous
import functools

import jax
import jax.numpy as jnp
from jax import lax
from jax.experimental import pallas as pl
from jax.experimental.pallas import tpu as pltpu
from jax.experimental.pallas import tpu_sc as plsc

GRID_N = 256
NPTS = 8192 * 128
NW = 32
PER_W = NPTS // NW
BC = 1024
NB = PER_W // BC
ROW = 128
NR = BC // ROW
L = 16
TABLE_W = 9 * 65536
ZERO_W = 8 * 65536
MAGIC = 12582912.0


def _pack_body(c_ref, out_ref):
    g = pl.program_id(0)
    x = c_ref[...].astype(jnp.int32)
    b = lax.broadcasted_iota(jnp.int32, (1, 32, 1, 1), 1)
    s = jnp.sum(x << b, axis=1)
    out_ref[...] = jnp.where(g < 8, s, 0)


def _pack(comb4d):
    return pl.pallas_call(
        _pack_body,
        grid=(9,),
        in_specs=[
            pl.BlockSpec((1, 32, GRID_N, GRID_N),
                         lambda g: (jnp.minimum(g, 7), 0, 0, 0)),
        ],
        out_specs=pl.BlockSpec((1, GRID_N, GRID_N), lambda g: (g, 0, 0)),
        out_shape=jax.ShapeDtypeStruct((9, GRID_N, GRID_N), jnp.int32),
    )(comb4d)


def _sc_body(xyz_hbm, table_hbm, params_hbm, out_hbm,
             pv, xall, wv, av, gv, ov, tsh, insem, gsem):
    sid = lax.axis_index("s")
    wid = sid * 2 + lax.axis_index("c")
    base_pt = wid * PER_W

    TW = TABLE_W // 16
    pltpu.sync_copy(table_hbm.at[pl.ds(sid * TW, TW)],
                    tsh.at[pl.ds(sid * TW, TW)])
    plsc.subcore_barrier()

    pltpu.sync_copy(params_hbm, pv)
    sx = pv[pl.ds(0, L)]
    sy = pv[pl.ds(L, L)]
    sz = pv[pl.ds(2 * L, L)]
    tx = pv[pl.ds(3 * L, L)]
    ty = pv[pl.ds(4 * L, L)]
    tz = pv[pl.ds(5 * L, L)]

    for c in range(3):
        pltpu.async_copy(xyz_hbm.at[pl.ds(c * NPTS + base_pt, BC)],
                         xall.at[pl.ds(c * BC, BC)], insem)

    def block_body(t, carry):
        pt0 = base_pt + t * BC
        pltpu.make_async_copy(xyz_hbm.at[pl.ds(0, 3 * BC)], xall, insem).wait()

        for r in range(NR):
            for gg in range(ROW // L):
                off = r * ROW + gg * L
                xs = xall[pl.ds(off, L)]
                ys = xall[pl.ds(BC + off, L)]
                zs = xall[pl.ds(2 * BC + off, L)]

                fx = (xs * sx + tx + MAGIC) - MAGIC
                fy = (ys * sy + ty + MAGIC) - MAGIC
                fz = (zs * sz + tz + MAGIC) - MAGIC

                ii = fx.astype(jnp.int32)
                jj = fy.astype(jnp.int32)
                kk = fz.astype(jnp.int32)

                oob = (ii | jj | kk) & -256
                w0 = (((ii & 224) << 11) | ((jj & 255) << 8)
                      | (kk & 255))
                wv[pl.ds(off, L)] = jnp.where(oob == 0, w0, ZERO_W)
                av[pl.ds(off, L)] = ii & 31

            pltpu.async_copy(tsh.at[wv.at[pl.ds(r * ROW, ROW)]],
                             gv.at[pl.ds(r * ROW, ROW)], gsem)

        ptn = base_pt + lax.rem(t + 1, NB) * BC
        for c in range(3):
            pltpu.async_copy(xyz_hbm.at[pl.ds(c * NPTS + ptn, BC)],
                             xall.at[pl.ds(c * BC, BC)], insem)

        pltpu.make_async_copy(table_hbm.at[pl.ds(0, BC)], gv, gsem).wait()

        for g in range(BC // L):
            w = gv[pl.ds(g * L, L)]
            a = av[pl.ds(g * L, L)]
            ov[pl.ds(g * L, L)] = lax.shift_right_logical(w, a) & 1

        pltpu.sync_copy(ov, out_hbm.at[pl.ds(pt0, BC)])
        return carry

    lax.fori_loop(0, NB, block_body, 0)
    pltpu.make_async_copy(xyz_hbm.at[pl.ds(0, 3 * BC)], xall, insem).wait()


@functools.partial(
    pl.kernel,
    out_type=jax.ShapeDtypeStruct((NPTS,), jnp.int32),
    mesh=plsc.VectorSubcoreMesh(core_axis_name="c", subcore_axis_name="s"),
    compiler_params=pltpu.CompilerParams(needs_layout_passes=False),
    scratch_types=[
        pltpu.VMEM((6 * L,), jnp.float32),
        pltpu.VMEM((3 * BC,), jnp.float32),
        pltpu.VMEM((BC,), jnp.int32),
        pltpu.VMEM((BC,), jnp.int32),
        pltpu.VMEM((BC,), jnp.int32),
        pltpu.VMEM((BC,), jnp.int32),
        pltpu.VMEM_SHARED((TABLE_W,), jnp.int32),
        pltpu.SemaphoreType.DMA,
        pltpu.SemaphoreType.DMA,
    ],
)
def _lookup(xyz_hbm, table_hbm, params_hbm, out_hbm, *scratch):
    _sc_body(xyz_hbm, table_hbm, params_hbm, out_hbm, *scratch)


def kernel(xyz, mask, bound_mask, xyz2ijk_scale, xyz2ijk_shift):
    shape = xyz.shape[:-1]
    comb_u8 = jnp.logical_and(mask, bound_mask).astype(jnp.uint8)
    packed = _pack(comb_u8.reshape(8, 32, GRID_N, GRID_N)).reshape(-1)
    xflat = jnp.transpose(xyz, (2, 0, 1)).reshape(-1)
    params = jnp.concatenate([
        jnp.repeat(xyz2ijk_scale.astype(jnp.float32), L),
        jnp.repeat(xyz2ijk_shift.astype(jnp.float32), L),
    ])
    flat = _lookup(xflat, packed, params)
    return flat.astype(jnp.bool_).reshape(shape)


# speedup vs baseline: 2.3159x; 1.3641x over previous
import functools

import jax
import jax.numpy as jnp
from jax import lax
from jax.experimental import pallas as pl
from jax.experimental.pallas import tpu as pltpu
from jax.experimental.pallas import tpu_sc as plsc

GRID_N = 256
NPTS = 8192 * 128
NW = 32
PER_W = NPTS // NW
BC = 1024
NB = PER_W // BC
ROW = 128
NR = BC // ROW
L = 16
TABLE_W = 9 * 65536
ZERO_W = 8 * 65536
MAGIC = 12582912.0


def _pack_body(c_ref, out_ref):
    g = pl.program_id(0)
    x = c_ref[...].astype(jnp.int32)
    b = lax.broadcasted_iota(jnp.int32, (1, 32, 1, 1), 1)
    s = jnp.sum(x << b, axis=1)
    out_ref[...] = jnp.where(g < 8, s, 0)


def _pack(comb4d):
    return pl.pallas_call(
        _pack_body,
        grid=(9,),
        in_specs=[
            pl.BlockSpec((1, 32, GRID_N, GRID_N),
                         lambda g: (jnp.minimum(g, 7), 0, 0, 0)),
        ],
        out_specs=pl.BlockSpec((1, GRID_N, GRID_N), lambda g: (g, 0, 0)),
        out_shape=jax.ShapeDtypeStruct((9, GRID_N, GRID_N), jnp.int32),
    )(comb4d)


def _sc_body(xyz_hbm, table_hbm, params_hbm, out_hbm,
             pv, xall, wv, av, gv, ov, tsh, insem, gsem):
    sid = lax.axis_index("s")
    wid = sid * 2 + lax.axis_index("c")
    base_pt = wid * PER_W

    TW = TABLE_W // 16
    pltpu.sync_copy(table_hbm.at[pl.ds(sid * TW, TW)],
                    tsh.at[pl.ds(sid * TW, TW)])
    plsc.subcore_barrier()

    pltpu.sync_copy(params_hbm, pv)
    sx = pv[pl.ds(0, L)]
    sy = pv[pl.ds(L, L)]
    sz = pv[pl.ds(2 * L, L)]
    tx = pv[pl.ds(3 * L, L)]
    ty = pv[pl.ds(4 * L, L)]
    tz = pv[pl.ds(5 * L, L)]

    for c in range(3):
        pltpu.async_copy(xyz_hbm.at[pl.ds(c * NPTS + base_pt, BC)],
                         xall.at[pl.ds(c * BC, BC)], insem)

    def block_body(t, carry):
        pt0 = base_pt + t * BC
        pltpu.make_async_copy(xyz_hbm.at[pl.ds(0, 3 * BC)], xall, insem).wait()

        for r in range(NR):
            for gg in range(ROW // L):
                off = r * ROW + gg * L
                xs = xall[pl.ds(off, L)]
                ys = xall[pl.ds(BC + off, L)]
                zs = xall[pl.ds(2 * BC + off, L)]

                fx = (xs * sx + tx + MAGIC) - MAGIC
                fy = (ys * sy + ty + MAGIC) - MAGIC
                fz = (zs * sz + tz + MAGIC) - MAGIC

                ii = fx.astype(jnp.int32)
                jj = fy.astype(jnp.int32)
                kk = fz.astype(jnp.int32)

                oob = (ii | jj | kk) & -256
                w0 = (((ii & 224) << 11) | ((jj & 255) << 8)
                      | (kk & 255))
                wv[pl.ds(off, L)] = jnp.where(
                    oob == 0, w0, (w0 & 65535) | ZERO_W)
                av[pl.ds(off, L)] = ii & 31

            pltpu.async_copy(tsh.at[wv.at[pl.ds(r * ROW, ROW)]],
                             gv.at[pl.ds(r * ROW, ROW)], gsem)

        ptn = base_pt + lax.rem(t + 1, NB) * BC
        for c in range(3):
            pltpu.async_copy(xyz_hbm.at[pl.ds(c * NPTS + ptn, BC)],
                             xall.at[pl.ds(c * BC, BC)], insem)

        pltpu.make_async_copy(table_hbm.at[pl.ds(0, BC)], gv, gsem).wait()

        for g in range(BC // L):
            w = gv[pl.ds(g * L, L)]
            a = av[pl.ds(g * L, L)]
            ov[pl.ds(g * L, L)] = lax.shift_right_logical(w, a) & 1

        pltpu.sync_copy(ov, out_hbm.at[pl.ds(pt0, BC)])
        return carry

    lax.fori_loop(0, NB, block_body, 0)
    pltpu.make_async_copy(xyz_hbm.at[pl.ds(0, 3 * BC)], xall, insem).wait()


@functools.partial(
    pl.kernel,
    out_type=jax.ShapeDtypeStruct((NPTS,), jnp.int32),
    mesh=plsc.VectorSubcoreMesh(core_axis_name="c", subcore_axis_name="s"),
    compiler_params=pltpu.CompilerParams(needs_layout_passes=False),
    scratch_types=[
        pltpu.VMEM((6 * L,), jnp.float32),
        pltpu.VMEM((3 * BC,), jnp.float32),
        pltpu.VMEM((BC,), jnp.int32),
        pltpu.VMEM((BC,), jnp.int32),
        pltpu.VMEM((BC,), jnp.int32),
        pltpu.VMEM((BC,), jnp.int32),
        pltpu.VMEM_SHARED((TABLE_W,), jnp.int32),
        pltpu.SemaphoreType.DMA,
        pltpu.SemaphoreType.DMA,
    ],
)
def _lookup(xyz_hbm, table_hbm, params_hbm, out_hbm, *scratch):
    _sc_body(xyz_hbm, table_hbm, params_hbm, out_hbm, *scratch)


def kernel(xyz, mask, bound_mask, xyz2ijk_scale, xyz2ijk_shift):
    shape = xyz.shape[:-1]
    comb_u8 = jnp.logical_and(mask, bound_mask).astype(jnp.uint8)
    packed = _pack(comb_u8.reshape(8, 32, GRID_N, GRID_N)).reshape(-1)
    xflat = jnp.transpose(xyz, (2, 0, 1)).reshape(-1)
    params = jnp.concatenate([
        jnp.repeat(xyz2ijk_scale.astype(jnp.float32), L),
        jnp.repeat(xyz2ijk_shift.astype(jnp.float32), L),
    ])
    flat = _lookup(xflat, packed, params)
    return flat.astype(jnp.bool_).reshape(shape)
